# baseline (speedup 1.0000x reference)
import functools

import jax
import jax.numpy as jnp
import numpy as np
from jax import lax
from jax.experimental import pallas as pl
from jax.experimental.pallas import tpu as pltpu
from jax.experimental.pallas import tpu_sc as plsc

N_NODES = 10000
N_EDGES = 320000
DIM = 128

NC = 2
NS = 16
NW = NC * NS
CH = 128
CPW = 80
PADDED_CHUNKS = NW * CPW + 2
PADDED_E = PADDED_CHUNKS * CH
ROWS_PER_TILE = 624
TAIL_ROWS = N_NODES - NS * ROWS_PER_TILE

_PERM = np.empty((DIM,), np.int32)
for _j in range(DIM // 32):
    for _t in range(16):
        _PERM[32 * _j + 2 * _t] = 32 * _j + _t
        _PERM[32 * _j + 2 * _t + 1] = 32 * _j + 16 + _t


def _mm_body(x_ref, w_ref, b_ref, o_ref):
    o_ref[...] = (lax.dot_general(
        x_ref[...], w_ref[...], (((1,), (1,)), ((), ())),
        preferred_element_type=jnp.float32) + b_ref[...]).astype(jnp.bfloat16)


_matmul = pl.pallas_call(
    _mm_body,
    grid=(5,),
    in_specs=[
        pl.BlockSpec((2000, DIM), lambda i: (i, 0)),
        pl.BlockSpec((DIM, DIM), lambda i: (0, 0)),
        pl.BlockSpec((1, DIM), lambda i: (0, 0)),
    ],
    out_specs=pl.BlockSpec((2000, DIM), lambda i: (i, 0)),
    out_shape=jax.ShapeDtypeStruct((N_NODES, DIM), jnp.bfloat16),
)


def _add_body(p_ref, o_ref):
    o_ref[...] = p_ref[0] + p_ref[1]


_combine = pl.pallas_call(
    _add_body,
    grid=(10,),
    in_specs=[pl.BlockSpec((2, 1000, DIM), lambda i: (0, i, 0))],
    out_specs=pl.BlockSpec((1000, DIM), lambda i: (i, 0)),
    out_shape=jax.ShapeDtypeStruct((N_NODES, DIM), jnp.float32),
)


_MESH = plsc.VectorSubcoreMesh(
    core_axis_name="c", subcore_axis_name="s", num_cores=NC, num_subcores=NS)


@functools.partial(
    pl.kernel,
    out_type=jax.ShapeDtypeStruct((NC, N_NODES, DIM), jnp.float32),
    mesh=_MESH,
    compiler_params=pltpu.CompilerParams(
        needs_layout_passes=False, use_tc_tiling_on_sc=False),
    scratch_types=[
        pltpu.VMEM((4, CH), jnp.int32),
        pltpu.VMEM((4, CH), jnp.int32),
        pltpu.VMEM((4, CH), jnp.float32),
        pltpu.VMEM((2, CH, DIM // 2), jnp.int32),
        pltpu.VMEM((CH, DIM), jnp.float32),
        pltpu.VMEM_SHARED((N_NODES, DIM), jnp.float32),
        pltpu.SemaphoreType.DMA((4,)),
        pltpu.SemaphoreType.DMA((2,)),
    ],
)
def _spmm(h_hbm, row_hbm, col_hbm, vals_hbm, out_hbm,
          col_v, row_v, vals_v, rows_bf, rows_f, acc, isem, gsem):
    cid = lax.axis_index("c")
    sid = lax.axis_index("s")
    wid = sid * NC + cid

    zv = jnp.zeros((16,), jnp.float32)

    def _zero_body(e, carry):
        for f in range(DIM // 16):
            rows_f[e, pl.ds(f * 16, 16)] = zv
        return carry

    lax.fori_loop(0, CH, _zero_body, 0)
    r0 = sid * ROWS_PER_TILE
    for j in range(4):
        pltpu.sync_copy(rows_f, acc.at[pl.ds(r0 + j * CH, CH)])
    pltpu.sync_copy(rows_f.at[pl.ds(0, ROWS_PER_TILE - 4 * CH)],
                    acc.at[pl.ds(r0 + 4 * CH, ROWS_PER_TILE - 4 * CH)])

    @pl.when(sid == NS - 1)
    def _zero_tail():
        pltpu.sync_copy(rows_f.at[pl.ds(0, TAIL_ROWS)],
                        acc.at[pl.ds(NS * ROWS_PER_TILE, TAIL_ROWS)])

    plsc.subcore_barrier()

    def chunk_base(c):
        return (wid * CPW + c) * CH

    def issue_idx(c, islot):
        base = chunk_base(c)
        pltpu.async_copy(row_hbm.at[pl.ds(base, CH)], row_v.at[islot],
                         isem.at[islot])
        pltpu.async_copy(col_hbm.at[pl.ds(base, CH)], col_v.at[islot],
                         isem.at[islot])
        pltpu.async_copy(vals_hbm.at[pl.ds(base, CH)], vals_v.at[islot],
                         isem.at[islot])

    def wait_idx(islot):
        pltpu.make_async_copy(row_hbm.at[pl.ds(0, CH)], row_v.at[islot],
                              isem.at[islot]).wait()
        pltpu.make_async_copy(col_hbm.at[pl.ds(0, CH)], col_v.at[islot],
                              isem.at[islot]).wait()
        pltpu.make_async_copy(vals_hbm.at[pl.ds(0, CH)], vals_v.at[islot],
                              isem.at[islot]).wait()

    def issue_gather(rslot, islot):
        pltpu.async_copy(h_hbm.at[col_v.at[islot]], rows_bf.at[rslot],
                         gsem.at[rslot])

    def wait_gather(rslot):
        pltpu.make_async_copy(h_hbm.at[col_v.at[0]], rows_bf.at[rslot],
                              gsem.at[rslot]).wait()

    def scale(rslot, islot):
        himask = jnp.full((16,), -65536, jnp.int32)

        @plsc.parallel_loop(0, CH, 1, unroll=8)
        def _scale_body(e):
            valv = plsc.load_gather(vals_v.at[islot],
                                    [jnp.full((16,), e, jnp.int32)])
            for j in range(DIM // 32):
                packed = rows_bf[rslot, e, pl.ds(j * 16, 16)]
                lo = plsc.bitcast(packed << 16, jnp.float32)
                hi = plsc.bitcast(packed & himask, jnp.float32)
                rows_f[e, pl.ds(j * 32, 16)] = lo * valv
                rows_f[e, pl.ds(j * 32 + 16, 16)] = hi * valv

    def scatter(islot):
        pltpu.sync_copy(rows_f, acc.at[row_v.at[islot]], add=True)

    issue_idx(0, 0)
    issue_idx(1, 1)
    wait_idx(0)
    issue_gather(0, 0)

    def _outer(j, carry):
        for k in range(4):
            c = j * 4 + k
            islot, rslot = k, k % 2
            issue_idx(c + 2, (k + 2) % 4)
            wait_idx((k + 1) % 4)
            issue_gather((k + 1) % 2, (k + 1) % 4)
            wait_gather(rslot)
            scale(rslot, islot)
            scatter(islot)
        return carry

    lax.fori_loop(0, CPW // 4, _outer, 0)

    wait_idx(1)
    wait_gather(0)

    plsc.subcore_barrier()
    pltpu.sync_copy(acc.at[pl.ds(r0, ROWS_PER_TILE)],
                    out_hbm.at[cid, pl.ds(r0, ROWS_PER_TILE)])

    @pl.when(sid == NS - 1)
    def _write_tail():
        pltpu.sync_copy(acc.at[pl.ds(NS * ROWS_PER_TILE, TAIL_ROWS)],
                        out_hbm.at[cid, pl.ds(NS * ROWS_PER_TILE, TAIL_ROWS)])


def kernel(x, adj_indices, adj_values, W, b):
    idx = adj_indices.astype(jnp.int32)
    pad = PADDED_E - N_EDGES
    spread = jnp.arange(pad, dtype=jnp.int32) % N_NODES
    row = jnp.concatenate([idx[0], spread])
    col = jnp.concatenate([idx[1], spread])
    vals = jnp.concatenate([adj_values, jnp.zeros((pad,), jnp.float32)])
    perm = jnp.asarray(_PERM)
    h = _matmul(x, W[perm], b[perm].reshape(1, DIM))
    h_packed = lax.bitcast_convert_type(
        h.reshape(N_NODES, DIM // 2, 2), jnp.int32)
    parts = _spmm(h_packed, row, col, vals)
    return _combine(parts)


# speedup vs baseline: 1.0920x; 1.0920x over previous
import functools

import jax
import jax.numpy as jnp
from jax import lax
from jax.experimental import pallas as pl
from jax.experimental.pallas import tpu as pltpu
from jax.experimental.pallas import tpu_sc as plsc

N_NODES = 10000
N_EDGES = 320000
DIM = 128

NC = 2
NS = 16
NW = NC * NS
CH = 128
CPW = 80
PADDED_CHUNKS = NW * CPW + 2
PADDED_E = PADDED_CHUNKS * CH
ROWS_PER_TILE = 624
TAIL_ROWS = N_NODES - NS * ROWS_PER_TILE


def _mm_body(x_ref, w_ref, b_ref, o_ref):
    o_ref[...] = lax.dot_general(
        x_ref[...], w_ref[...], (((1,), (1,)), ((), ())),
        preferred_element_type=jnp.float32) + b_ref[...]


_matmul = pl.pallas_call(
    _mm_body,
    grid=(10,),
    in_specs=[
        pl.BlockSpec((1000, DIM), lambda i: (i, 0)),
        pl.BlockSpec((DIM, DIM), lambda i: (0, 0)),
        pl.BlockSpec((1, DIM), lambda i: (0, 0)),
    ],
    out_specs=pl.BlockSpec((1000, DIM), lambda i: (i, 0)),
    out_shape=jax.ShapeDtypeStruct((N_NODES, DIM), jnp.float32),
)


def _add_body(p_ref, o_ref):
    o_ref[...] = p_ref[0] + p_ref[1]


_combine = pl.pallas_call(
    _add_body,
    grid=(10,),
    in_specs=[pl.BlockSpec((2, 1000, DIM), lambda i: (0, i, 0))],
    out_specs=pl.BlockSpec((1000, DIM), lambda i: (i, 0)),
    out_shape=jax.ShapeDtypeStruct((N_NODES, DIM), jnp.float32),
)


_MESH = plsc.VectorSubcoreMesh(
    core_axis_name="c", subcore_axis_name="s", num_cores=NC, num_subcores=NS)


@functools.partial(
    pl.kernel,
    out_type=jax.ShapeDtypeStruct((NC, N_NODES, DIM), jnp.float32),
    mesh=_MESH,
    compiler_params=pltpu.CompilerParams(needs_layout_passes=False),
    scratch_types=[
        pltpu.VMEM((4, CH), jnp.int32),
        pltpu.VMEM((4, CH), jnp.int32),
        pltpu.VMEM((4, CH), jnp.float32),
        pltpu.VMEM((2, CH, DIM), jnp.float32),
        pltpu.VMEM_SHARED((N_NODES, DIM), jnp.float32),
        pltpu.SemaphoreType.DMA((4,)),
        pltpu.SemaphoreType.DMA((2,)),
    ],
)
def _spmm(h_hbm, row_hbm, col_hbm, vals_hbm, out_hbm,
          col_v, row_v, vals_v, rows_v, acc, isem, gsem):
    cid = lax.axis_index("c")
    sid = lax.axis_index("s")
    wid = sid * NC + cid

    zv = jnp.zeros((16,), jnp.float32)

    def _zero_body(e, carry):
        for f in range(DIM // 16):
            rows_v[0, e, pl.ds(f * 16, 16)] = zv
        return carry

    lax.fori_loop(0, CH, _zero_body, 0)
    r0 = sid * ROWS_PER_TILE
    for j in range(4):
        pltpu.sync_copy(rows_v.at[0], acc.at[pl.ds(r0 + j * CH, CH)])
    pltpu.sync_copy(rows_v.at[0, pl.ds(0, ROWS_PER_TILE - 4 * CH)],
                    acc.at[pl.ds(r0 + 4 * CH, ROWS_PER_TILE - 4 * CH)])

    @pl.when(sid == NS - 1)
    def _zero_tail():
        pltpu.sync_copy(rows_v.at[0, pl.ds(0, TAIL_ROWS)],
                        acc.at[pl.ds(NS * ROWS_PER_TILE, TAIL_ROWS)])

    plsc.subcore_barrier()

    def chunk_base(c):
        return (wid * CPW + c) * CH

    def issue_idx(c, islot):
        base = chunk_base(c)
        pltpu.async_copy(row_hbm.at[pl.ds(base, CH)], row_v.at[islot],
                         isem.at[islot])
        pltpu.async_copy(col_hbm.at[pl.ds(base, CH)], col_v.at[islot],
                         isem.at[islot])
        pltpu.async_copy(vals_hbm.at[pl.ds(base, CH)], vals_v.at[islot],
                         isem.at[islot])

    def wait_idx(islot):
        pltpu.make_async_copy(row_hbm.at[pl.ds(0, CH)], row_v.at[islot],
                              isem.at[islot]).wait()
        pltpu.make_async_copy(col_hbm.at[pl.ds(0, CH)], col_v.at[islot],
                              isem.at[islot]).wait()
        pltpu.make_async_copy(vals_hbm.at[pl.ds(0, CH)], vals_v.at[islot],
                              isem.at[islot]).wait()

    def issue_gather(rslot, islot):
        pltpu.async_copy(h_hbm.at[col_v.at[islot]], rows_v.at[rslot],
                         gsem.at[rslot])

    def wait_gather(rslot):
        pltpu.make_async_copy(h_hbm.at[col_v.at[0]], rows_v.at[rslot],
                              gsem.at[rslot]).wait()

    def scale(rslot, islot):
        @plsc.parallel_loop(0, CH, 1, unroll=8)
        def _scale_body(e):
            valv = plsc.load_gather(vals_v.at[islot],
                                    [jnp.full((16,), e, jnp.int32)])
            for f in range(DIM // 16):
                sl = pl.ds(f * 16, 16)
                rows_v[rslot, e, sl] = rows_v[rslot, e, sl] * valv

    def scatter(rslot, islot):
        pltpu.sync_copy(rows_v.at[rslot], acc.at[row_v.at[islot]], add=True)

    issue_idx(0, 0)
    issue_idx(1, 1)
    wait_idx(0)
    issue_gather(0, 0)

    def _outer(j, carry):
        for k in range(4):
            c = j * 4 + k
            islot, rslot = k, k % 2
            issue_idx(c + 2, (k + 2) % 4)
            wait_idx((k + 1) % 4)
            issue_gather((k + 1) % 2, (k + 1) % 4)
            wait_gather(rslot)
            scale(rslot, islot)
            scatter(rslot, islot)
        return carry

    lax.fori_loop(0, CPW // 4, _outer, 0)

    wait_idx(1)
    wait_gather(0)

    plsc.subcore_barrier()
    pltpu.sync_copy(acc.at[pl.ds(r0, ROWS_PER_TILE)],
                    out_hbm.at[cid, pl.ds(r0, ROWS_PER_TILE)])

    @pl.when(sid == NS - 1)
    def _write_tail():
        pltpu.sync_copy(acc.at[pl.ds(NS * ROWS_PER_TILE, TAIL_ROWS)],
                        out_hbm.at[cid, pl.ds(NS * ROWS_PER_TILE, TAIL_ROWS)])


def kernel(x, adj_indices, adj_values, W, b):
    idx = adj_indices.astype(jnp.int32)
    pad = PADDED_E - N_EDGES
    spread = jnp.arange(pad, dtype=jnp.int32) % N_NODES
    row = jnp.concatenate([idx[0], spread])
    col = jnp.concatenate([idx[1], spread])
    vals = jnp.concatenate([adj_values, jnp.zeros((pad,), jnp.float32)])
    h = _matmul(x, W, b.reshape(1, DIM))
    parts = _spmm(h, row, col, vals)
    return _combine(parts)


# speedup vs baseline: 1.1027x; 1.0098x over previous
import functools

import jax
import jax.numpy as jnp
from jax import lax
from jax.experimental import pallas as pl
from jax.experimental.pallas import tpu as pltpu
from jax.experimental.pallas import tpu_sc as plsc

N_NODES = 10000
N_EDGES = 320000
DIM = 128

NC = 2
NS = 16
NW = NC * NS
CH = 128
CPW = 80
PADDED_CHUNKS = NW * CPW + 2
PADDED_E = PADDED_CHUNKS * CH
ROWS_PER_TILE = 624
TAIL_ROWS = N_NODES - NS * ROWS_PER_TILE


def _mm_body(x_ref, w_ref, b_ref, o_ref):
    o_ref[...] = lax.dot_general(
        x_ref[...], w_ref[...], (((1,), (1,)), ((), ())),
        preferred_element_type=jnp.float32) + b_ref[...]


_matmul = pl.pallas_call(
    _mm_body,
    grid=(10,),
    in_specs=[
        pl.BlockSpec((1000, DIM), lambda i: (i, 0)),
        pl.BlockSpec((DIM, DIM), lambda i: (0, 0)),
        pl.BlockSpec((1, DIM), lambda i: (0, 0)),
    ],
    out_specs=pl.BlockSpec((1000, DIM), lambda i: (i, 0)),
    out_shape=jax.ShapeDtypeStruct((N_NODES, DIM), jnp.float32),
)


def _add_body(p_ref, o_ref):
    o_ref[...] = p_ref[0] + p_ref[1]


_combine = pl.pallas_call(
    _add_body,
    grid=(10,),
    in_specs=[pl.BlockSpec((2, 1000, DIM), lambda i: (0, i, 0))],
    out_specs=pl.BlockSpec((1000, DIM), lambda i: (i, 0)),
    out_shape=jax.ShapeDtypeStruct((N_NODES, DIM), jnp.float32),
)


_MESH = plsc.VectorSubcoreMesh(
    core_axis_name="c", subcore_axis_name="s", num_cores=NC, num_subcores=NS)


@functools.partial(
    pl.kernel,
    out_type=jax.ShapeDtypeStruct((NC, N_NODES, DIM), jnp.float32),
    mesh=_MESH,
    compiler_params=pltpu.CompilerParams(needs_layout_passes=False),
    scratch_types=[
        pltpu.VMEM((4, CH), jnp.int32),
        pltpu.VMEM((4, CH), jnp.int32),
        pltpu.VMEM((4, CH), jnp.float32),
        pltpu.VMEM((2, CH, DIM), jnp.float32),
        pltpu.VMEM_SHARED((N_NODES, DIM), jnp.float32),
        pltpu.SemaphoreType.DMA((4,)),
        pltpu.SemaphoreType.DMA((2,)),
    ],
)
def _spmm(h_hbm, row_hbm, col_hbm, vals_hbm, out_hbm,
          col_v, row_v, vals_v, rows_v, acc, isem, gsem):
    cid = lax.axis_index("c")
    sid = lax.axis_index("s")
    wid = sid * NC + cid

    def chunk_base(c):
        return (wid * CPW + c) * CH

    def issue_idx(c, islot):
        base = chunk_base(c)
        pltpu.async_copy(row_hbm.at[pl.ds(base, CH)], row_v.at[islot],
                         isem.at[islot])
        pltpu.async_copy(col_hbm.at[pl.ds(base, CH)], col_v.at[islot],
                         isem.at[islot])
        pltpu.async_copy(vals_hbm.at[pl.ds(base, CH)], vals_v.at[islot],
                         isem.at[islot])

    def wait_idx(islot):
        pltpu.make_async_copy(row_hbm.at[pl.ds(0, CH)], row_v.at[islot],
                              isem.at[islot]).wait()
        pltpu.make_async_copy(col_hbm.at[pl.ds(0, CH)], col_v.at[islot],
                              isem.at[islot]).wait()
        pltpu.make_async_copy(vals_hbm.at[pl.ds(0, CH)], vals_v.at[islot],
                              isem.at[islot]).wait()

    def issue_gather(rslot, islot):
        pltpu.async_copy(h_hbm.at[col_v.at[islot]], rows_v.at[rslot],
                         gsem.at[rslot])

    def wait_gather(rslot):
        pltpu.make_async_copy(h_hbm.at[col_v.at[0]], rows_v.at[rslot],
                              gsem.at[rslot]).wait()

    def scale(rslot, islot):
        @plsc.parallel_loop(0, CH, 1, unroll=8)
        def _scale_body(e):
            valv = plsc.load_gather(vals_v.at[islot],
                                    [jnp.full((16,), e, jnp.int32)])
            for f in range(DIM // 16):
                sl = pl.ds(f * 16, 16)
                rows_v[rslot, e, sl] = rows_v[rslot, e, sl] * valv

    def scatter(rslot, islot):
        pltpu.sync_copy(rows_v.at[rslot], acc.at[row_v.at[islot]], add=True)

    issue_idx(0, 0)
    issue_idx(1, 1)
    wait_idx(0)
    issue_gather(0, 0)

    zv = jnp.zeros((16,), jnp.float32)

    def _zero_body(e, carry):
        for f in range(DIM // 16):
            rows_v[1, e, pl.ds(f * 16, 16)] = zv
        return carry

    lax.fori_loop(0, CH, _zero_body, 0)
    r0 = sid * ROWS_PER_TILE
    for j in range(4):
        pltpu.sync_copy(rows_v.at[1], acc.at[pl.ds(r0 + j * CH, CH)])
    pltpu.sync_copy(rows_v.at[1, pl.ds(0, ROWS_PER_TILE - 4 * CH)],
                    acc.at[pl.ds(r0 + 4 * CH, ROWS_PER_TILE - 4 * CH)])

    @pl.when(sid == NS - 1)
    def _zero_tail():
        pltpu.sync_copy(rows_v.at[1, pl.ds(0, TAIL_ROWS)],
                        acc.at[pl.ds(NS * ROWS_PER_TILE, TAIL_ROWS)])

    plsc.subcore_barrier()

    def _outer(j, carry):
        for k in range(4):
            c = j * 4 + k
            islot, rslot = k, k % 2
            issue_idx(c + 2, (k + 2) % 4)
            wait_idx((k + 1) % 4)
            issue_gather((k + 1) % 2, (k + 1) % 4)
            wait_gather(rslot)
            scale(rslot, islot)
            scatter(rslot, islot)
        return carry

    lax.fori_loop(0, CPW // 4, _outer, 0)

    wait_idx(1)
    wait_gather(0)

    plsc.subcore_barrier()
    pltpu.sync_copy(acc.at[pl.ds(r0, ROWS_PER_TILE)],
                    out_hbm.at[cid, pl.ds(r0, ROWS_PER_TILE)])

    @pl.when(sid == NS - 1)
    def _write_tail():
        pltpu.sync_copy(acc.at[pl.ds(NS * ROWS_PER_TILE, TAIL_ROWS)],
                        out_hbm.at[cid, pl.ds(NS * ROWS_PER_TILE, TAIL_ROWS)])


def kernel(x, adj_indices, adj_values, W, b):
    idx = adj_indices.astype(jnp.int32)
    pad = PADDED_E - N_EDGES
    spread = jnp.arange(pad, dtype=jnp.int32) % N_NODES
    row = jnp.concatenate([idx[0], spread])
    col = jnp.concatenate([idx[1], spread])
    vals = jnp.concatenate([adj_values, jnp.zeros((pad,), jnp.float32)])
    h = _matmul(x, W, b.reshape(1, DIM))
    parts = _spmm(h, row, col, vals)
    return _combine(parts)
